# baseline (speedup 1.0000x reference)
import functools

import jax
import jax.numpy as jnp
from jax import lax
from jax.experimental import pallas as pl
from jax.experimental.pallas import tpu as pltpu
from jax.experimental.pallas import tpu_sc as plsc

NC = 2
NS = 16
NW = NC * NS
CH = 128
L = 16


def _splat(scalar_i32, e):
    return jnp.zeros((L,), jnp.int32) + scalar_i32 + e


def _make_deg_kernel(n, nchunk):
    mesh = plsc.VectorSubcoreMesh(core_axis_name="c", subcore_axis_name="s",
                                  num_cores=NC, num_subcores=NS)

    def body(col_hbm, ew_hbm, zn_hbm, deg_hbm, col_v, ew_v, deg_sh):
        cid = lax.axis_index("c")
        sid = lax.axis_index("s")
        wid = cid * NS + sid
        pltpu.sync_copy(col_hbm.at[wid], col_v)
        pltpu.sync_copy(ew_hbm.at[wid], ew_v)

        @pl.when(sid == 0)
        def _():
            pltpu.sync_copy(zn_hbm, deg_sh)

        plsc.subcore_barrier()

        def step(j, carry):
            pltpu.sync_copy(ew_v.at[j], deg_sh.at[col_v.at[j]], add=True)
            return carry

        lax.fori_loop(0, nchunk, step, 0)
        plsc.subcore_barrier()

        @pl.when(sid == 0)
        def _():
            pltpu.sync_copy(deg_sh, deg_hbm.at[cid, 0])

    return pl.kernel(
        body,
        out_type=jax.ShapeDtypeStruct((NC, 1, n), jnp.float32),
        mesh=mesh,
        scratch_types=[
            pltpu.VMEM((nchunk, CH), jnp.int32),
            pltpu.VMEM((nchunk, CH), jnp.float32),
            pltpu.VMEM_SHARED((n,), jnp.float32),
        ],
    )


def _make_agg_kernel(n, d, nchunk, cha):
    mesh = plsc.VectorSubcoreMesh(core_axis_name="c", subcore_axis_name="s",
                                  num_cores=NC, num_subcores=NS)
    npad = -(-n // (8 * NS)) * 8 * NS
    rpt = npad // NS

    nstage = nchunk + 2

    NB = 3

    REC = 256

    def body(hs_hbm, recf_hbm, ewf_hbm, zn_hbm, out_hbm,
             recw, eww, scat_idx, rows_v,
             sr0, sr1, sr2, sg0, sg1, sg2, ss0, ss1, ss2,
             agg_sh):
        cid = lax.axis_index("c")
        sid = lax.axis_index("s")
        wid = cid * NS + sid
        base = wid * (nstage * REC)
        sl_init = pl.ds(sid * rpt, rpt)
        pltpu.sync_copy(zn_hbm.at[sl_init], agg_sh.at[sl_init])

        sr = (sr0, sr1, sr2)
        sg = (sg0, sg1, sg2)
        ss = (ss0, ss1, ss2)

        def idx_issue(k, r):
            sl = pl.ds(base + k * REC, REC)
            sle = pl.ds(wid * (nstage * cha) + k * cha, cha)
            pltpu.async_copy(recf_hbm.at[sl], recw.at[r, 0], sr[r])
            pltpu.async_copy(ewf_hbm.at[sle], eww.at[r], sr[r])

        def idx_wait(k, r):
            sl = pl.ds(base + k * REC, REC)
            sle = pl.ds(wid * (nstage * cha) + k * cha, cha)
            pltpu.make_async_copy(recf_hbm.at[sl], recw.at[r, 0],
                                  sr[r]).wait()
            pltpu.make_async_copy(ewf_hbm.at[sle], eww.at[r], sr[r]).wait()

        def row_ref(r):
            return recw.at[r, 0, pl.ds(0, cha)]

        def g_issue(r):
            pltpu.async_copy(hs_hbm.at[row_ref(r)], rows_v.at[r], sg[r])

        def g_wait(r):
            pltpu.make_async_copy(hs_hbm.at[row_ref(r)], rows_v.at[r],
                                  sg[r]).wait()

        def wait_scatter(r):
            pltpu.make_async_copy(rows_v.at[r], agg_sh.at[scat_idx.at[r]],
                                  ss[r]).wait()

        def scale(b):
            for g in range(cha // L):
                sl = pl.ds(g * L, L)
                scat_idx[b, sl] = recw[b, 0, pl.ds(128 + g * L, L)]

            for g in range(cha // L):
                nv = eww[b, pl.ds(g * L, L)]
                for i in range(L):
                    e = g * L + i
                    s = lax.gather(
                        nv, jnp.full((L, 1), i, jnp.int32),
                        lax.GatherDimensionNumbers(
                            offset_dims=(), collapsed_slice_dims=(0,),
                            start_index_map=(0,)),
                        (1,),
                        mode=lax.GatherScatterMode.PROMISE_IN_BOUNDS)
                    for fb in range(d // L):
                        sl = pl.ds(fb * L, L)
                        rows_v[b, e, sl] = rows_v[b, e, sl] * s

        idx_issue(0, 0)
        idx_issue(1, 1)
        idx_wait(0, 0)
        g_issue(0)
        plsc.subcore_barrier()

        def step(j, carry):
            for b in range(NB):
                k = NB * j + b
                nxt = (b + 1) % NB
                nn = (b + 2) % NB

                @pl.when((j > 0) | (b == 2))
                def _():
                    wait_scatter(nxt)

                idx_wait(k + 1, nxt)
                g_issue(nxt)
                idx_issue(k + 2, nn)
                g_wait(b)
                scale(b)
                pltpu.async_copy(rows_v.at[b], agg_sh.at[scat_idx.at[b]],
                                 ss[b], add=True)
            return carry

        lax.fori_loop(0, nchunk // NB, step, 0)
        wait_scatter((nchunk - 2) % NB)
        wait_scatter((nchunk - 1) % NB)
        g_wait(nchunk % NB)
        idx_wait(nchunk + 1, (nchunk + 1) % NB)
        plsc.subcore_barrier()
        pltpu.sync_copy(agg_sh.at[sl_init], out_hbm.at[cid, sl_init])

    return pl.kernel(
        body,
        out_type=jax.ShapeDtypeStruct((NC, npad, d), jnp.float32),
        mesh=mesh,
        scratch_types=(
            [pltpu.VMEM((NB, 1, REC), jnp.int32),
             pltpu.VMEM((NB, cha), jnp.float32),
             pltpu.VMEM((NB, cha), jnp.int32),
             pltpu.VMEM((NB, cha, d), jnp.float32)]
            + [pltpu.SemaphoreType.DMA] * 9
            + [pltpu.VMEM_SHARED((npad, d), jnp.float32)]
        ),
    ), npad


def _tc_pre_body(x_ref, wt1_ref, degt_ref, dinv_ref, h1s_ref):
    deg = degt_ref[...]
    degsum = deg[:, 0:1] + deg[:, 1:2] + 1.0
    dinv = jnp.where(degsum > 0, lax.rsqrt(degsum), 0.0)
    dinv_ref[...] = dinv
    h = jnp.dot(x_ref[...], wt1_ref[...],
                preferred_element_type=jnp.float32,
                precision=lax.Precision.HIGHEST)
    h1s_ref[...] = h * dinv


def _tc_mid_body(agg_ref, h1s_ref, dinv_ref, b1_ref, wt2_ref, h2s_ref):
    n = h1s_ref.shape[0]
    dinv = dinv_ref[...]
    agg = agg_ref[0][:n] + agg_ref[1][:n] + h1s_ref[...]
    out1 = jnp.maximum(dinv * agg + b1_ref[...], 0.0)
    h2 = jnp.dot(out1, wt2_ref[...],
                 preferred_element_type=jnp.float32,
                 precision=lax.Precision.HIGHEST)
    h2s_ref[...] = h2 * dinv


def _tc_post_body(agg_ref, h2s_ref, dinv_ref, b2_ref, out_ref):
    n = h2s_ref.shape[0]
    out_ref[...] = (dinv_ref[...] * (agg_ref[0][:n] + agg_ref[1][:n]
                                     + h2s_ref[...]) + b2_ref[...])


CHA = 96


@functools.lru_cache(maxsize=None)
def _make_calls(n, d, e):
    epw = e // NW
    ncd = -(-epw // CH)
    nca = 3 * (-(-epw // (3 * CHA)))
    deg_call = _make_deg_kernel(n, ncd)
    agg_call, npad = _make_agg_kernel(n, d, nca, CHA)

    tc_pre = pl.pallas_call(
        _tc_pre_body,
        out_shape=(jax.ShapeDtypeStruct((n, 1), jnp.float32),
                   jax.ShapeDtypeStruct((n, d), jnp.float32)))
    tc_mid = pl.pallas_call(
        _tc_mid_body,
        out_shape=jax.ShapeDtypeStruct((n, d), jnp.float32))
    tc_post = pl.pallas_call(
        _tc_post_body,
        out_shape=jax.ShapeDtypeStruct((n, d), jnp.float32))
    return deg_call, agg_call, tc_pre, tc_mid, tc_post, ncd, nca, npad


def _pad_per_worker(a, slots, epw, padvals):
    return jnp.concatenate([a.reshape(NW, epw), padvals], axis=1)


def kernel(x, edge_index, edge_weight, W1, b1, W2, b2):
    n, d_in = x.shape
    e = edge_weight.shape[0]
    d = W1.shape[0]
    (deg_call, agg_call, tc_pre, tc_mid, tc_post,
     ncd, nca, npad) = _make_calls(n, d, e)

    epw = e // NW
    row = edge_index[0]
    col = edge_index[1]
    ew = edge_weight
    padd = ncd * CH - epw
    pada = (nca + 2) * CHA - epw
    pidxd = (jnp.arange(NW * padd, dtype=jnp.int32) % n).reshape(NW, padd)
    pidxa = (jnp.arange(NW * pada, dtype=jnp.int32) % n).reshape(NW, pada)
    zpad = jnp.zeros((NW, padd), jnp.float32)
    zpaa = jnp.zeros((NW, pada), jnp.float32)

    col_pd = _pad_per_worker(col, ncd, epw, pidxd).reshape(NW, ncd, CH)
    ew_pd = _pad_per_worker(ew, ncd, epw, zpad).reshape(NW, ncd, CH)

    nstage = nca + 2

    def _to128(a3):
        tail = jnp.zeros((NW, nstage, 128 - CHA), a3.dtype)
        return jnp.concatenate([a3, tail], axis=2)

    row3 = _to128(_pad_per_worker(row, nca, epw, pidxa)
                  .reshape(NW, nstage, CHA))
    col3 = _to128(_pad_per_worker(col, nca, epw, pidxa)
                  .reshape(NW, nstage, CHA))
    recf = jnp.concatenate([row3, col3], axis=2).reshape(-1)
    ewf = _pad_per_worker(ew, nca, epw, zpaa).reshape(-1)

    zn = jnp.zeros((n,), jnp.float32)
    znd = jnp.zeros((npad, d), jnp.float32)

    deg_part = deg_call(col_pd, ew_pd, zn)
    dinv, h1s = tc_pre(x, W1.T, deg_part.reshape(NC, n).T)
    agg1 = agg_call(h1s, recf, ewf, znd)
    h2s = tc_mid(agg1, h1s, dinv, b1.reshape(1, d), W2.T)
    agg2 = agg_call(h2s, recf, ewf, znd)
    out = tc_post(agg2, h2s, dinv, b2.reshape(1, d))
    return out


# speedup vs baseline: 1.0146x; 1.0146x over previous
import functools

import jax
import jax.numpy as jnp
from jax import lax
from jax.experimental import pallas as pl
from jax.experimental.pallas import tpu as pltpu
from jax.experimental.pallas import tpu_sc as plsc

NC = 2
NS = 16
NW = NC * NS
CH = 128
L = 16


def _make_deg_kernel(n, nchunk):
    mesh = plsc.VectorSubcoreMesh(core_axis_name="c", subcore_axis_name="s",
                                  num_cores=NC, num_subcores=NS)

    def body(col_hbm, ew_hbm, zn_hbm, deg_hbm, col_v, ew_v, deg_sh):
        cid = lax.axis_index("c")
        sid = lax.axis_index("s")
        wid = cid * NS + sid
        pltpu.sync_copy(col_hbm.at[wid], col_v)
        pltpu.sync_copy(ew_hbm.at[wid], ew_v)

        @pl.when(sid == 0)
        def _():
            pltpu.sync_copy(zn_hbm, deg_sh)

        plsc.subcore_barrier()

        def step(j, carry):
            pltpu.sync_copy(ew_v.at[j], deg_sh.at[col_v.at[j]], add=True)
            return carry

        lax.fori_loop(0, nchunk, step, 0)
        plsc.subcore_barrier()

        @pl.when(sid == 0)
        def _():
            pltpu.sync_copy(deg_sh, deg_hbm.at[cid, 0])

    return pl.kernel(
        body,
        out_type=jax.ShapeDtypeStruct((NC, 1, n), jnp.float32),
        mesh=mesh,
        scratch_types=[
            pltpu.VMEM((nchunk, CH), jnp.int32),
            pltpu.VMEM((nchunk, CH), jnp.float32),
            pltpu.VMEM_SHARED((n,), jnp.float32),
        ],
    )


def _make_agg_kernel(n, d, nchunk, cha):
    mesh = plsc.VectorSubcoreMesh(core_axis_name="c", subcore_axis_name="s",
                                  num_cores=NC, num_subcores=NS)
    npad = -(-n // (8 * NS)) * 8 * NS
    rpt = npad // NS

    nstage = nchunk + 2

    NB = 3

    def body(hs_hbm, rowf_hbm, colf_hbm, ewf_hbm, zn_hbm, out_hbm,
             roww, colw, eww, scat_idx, rows_v,
             sr0, sr1, sr2, sg0, sg1, sg2, sc0, sc1, sc2, ss0, ss1, ss2,
             agg_sh):
        cid = lax.axis_index("c")
        sid = lax.axis_index("s")
        wid = cid * NS + sid
        base = wid * (nstage * cha)
        sl_init = pl.ds(sid * rpt, rpt)
        pltpu.sync_copy(zn_hbm.at[sl_init], agg_sh.at[sl_init])

        sr = (sr0, sr1, sr2)
        sg = (sg0, sg1, sg2)
        sc = (sc0, sc1, sc2)
        ss = (ss0, ss1, ss2)

        def idx_issue(k, r):
            sl = pl.ds(base + k * cha, cha)
            pltpu.async_copy(rowf_hbm.at[sl], roww.at[r], sr[r])
            pltpu.async_copy(colf_hbm.at[sl], colw.at[r], sc[r])
            pltpu.async_copy(ewf_hbm.at[sl], eww.at[r], sc[r])

        def idx_wait(k, r):
            sl = pl.ds(base + k * cha, cha)
            pltpu.make_async_copy(rowf_hbm.at[sl], roww.at[r], sr[r]).wait()
            pltpu.make_async_copy(colf_hbm.at[sl], colw.at[r], sc[r]).wait()
            pltpu.make_async_copy(ewf_hbm.at[sl], eww.at[r], sc[r]).wait()

        def g_issue(r):
            pltpu.async_copy(hs_hbm.at[roww.at[r]], rows_v.at[r], sg[r])

        def g_wait(r):
            pltpu.make_async_copy(hs_hbm.at[roww.at[r]], rows_v.at[r],
                                  sg[r]).wait()

        def wait_scatter(r):
            pltpu.make_async_copy(rows_v.at[r], agg_sh.at[scat_idx.at[r]],
                                  ss[r]).wait()

        def scale(b):
            for g in range(cha // L):
                sl = pl.ds(g * L, L)
                scat_idx[b, sl] = colw[b, sl]

            for g in range(cha // L):
                nv = eww[b, pl.ds(g * L, L)]
                for i in range(L):
                    e = g * L + i
                    s = lax.gather(
                        nv, jnp.full((L, 1), i, jnp.int32),
                        lax.GatherDimensionNumbers(
                            offset_dims=(), collapsed_slice_dims=(0,),
                            start_index_map=(0,)),
                        (1,),
                        mode=lax.GatherScatterMode.PROMISE_IN_BOUNDS)
                    for fb in range(d // L):
                        sl = pl.ds(fb * L, L)
                        rows_v[b, e, sl] = rows_v[b, e, sl] * s

        idx_issue(0, 0)
        idx_issue(1, 1)
        idx_wait(0, 0)
        g_issue(0)
        plsc.subcore_barrier()

        def step(j, carry):
            for b in range(NB):
                k = NB * j + b
                nxt = (b + 1) % NB
                nn = (b + 2) % NB

                @pl.when((j > 0) | (b == 2))
                def _():
                    wait_scatter(nxt)

                idx_wait(k + 1, nxt)
                g_issue(nxt)
                idx_issue(k + 2, nn)
                g_wait(b)
                scale(b)
                pltpu.async_copy(rows_v.at[b], agg_sh.at[scat_idx.at[b]],
                                 ss[b], add=True)
            return carry

        lax.fori_loop(0, nchunk // NB, step, 0)
        wait_scatter((nchunk - 2) % NB)
        wait_scatter((nchunk - 1) % NB)
        g_wait(nchunk % NB)
        idx_wait(nchunk + 1, (nchunk + 1) % NB)
        plsc.subcore_barrier()
        pltpu.sync_copy(agg_sh.at[sl_init], out_hbm.at[cid, sl_init])

    return pl.kernel(
        body,
        out_type=jax.ShapeDtypeStruct((NC, npad, d), jnp.float32),
        mesh=mesh,
        scratch_types=(
            [pltpu.VMEM((NB, cha), jnp.int32),
             pltpu.VMEM((NB, cha), jnp.int32),
             pltpu.VMEM((NB, cha), jnp.float32),
             pltpu.VMEM((NB, cha), jnp.int32),
             pltpu.VMEM((NB, cha, d), jnp.float32)]
            + [pltpu.SemaphoreType.DMA] * 12
            + [pltpu.VMEM_SHARED((npad, d), jnp.float32)]
        ),
    ), npad


def _tc_pre_body(x_ref, wt1_ref, degt_ref, dinv_ref, h1s_ref):
    deg = degt_ref[...]
    degsum = deg[:, 0:1] + deg[:, 1:2] + 1.0
    dinv = jnp.where(degsum > 0, lax.rsqrt(degsum), 0.0)
    dinv_ref[...] = dinv
    h = jnp.dot(x_ref[...], wt1_ref[...],
                preferred_element_type=jnp.float32,
                precision=lax.Precision.HIGHEST)
    h1s_ref[...] = h * dinv


def _tc_mid_body(agg_ref, h1s_ref, dinv_ref, b1_ref, wt2_ref, h2s_ref):
    n = h1s_ref.shape[0]
    dinv = dinv_ref[...]
    agg = agg_ref[0][:n] + agg_ref[1][:n] + h1s_ref[...]
    out1 = jnp.maximum(dinv * agg + b1_ref[...], 0.0)
    h2 = jnp.dot(out1, wt2_ref[...],
                 preferred_element_type=jnp.float32,
                 precision=lax.Precision.HIGHEST)
    h2s_ref[...] = h2 * dinv


def _tc_post_body(agg_ref, h2s_ref, dinv_ref, b2_ref, out_ref):
    n = h2s_ref.shape[0]
    out_ref[...] = (dinv_ref[...] * (agg_ref[0][:n] + agg_ref[1][:n]
                                     + h2s_ref[...]) + b2_ref[...])


CHA = 96


@functools.lru_cache(maxsize=None)
def _make_calls(n, d, e):
    epw = e // NW
    ncd = -(-epw // CH)
    nca = 3 * (-(-epw // (3 * CHA)))
    deg_call = _make_deg_kernel(n, ncd)
    agg_call, npad = _make_agg_kernel(n, d, nca, CHA)

    tc_pre = pl.pallas_call(
        _tc_pre_body,
        out_shape=(jax.ShapeDtypeStruct((n, 1), jnp.float32),
                   jax.ShapeDtypeStruct((n, d), jnp.float32)))
    tc_mid = pl.pallas_call(
        _tc_mid_body,
        out_shape=jax.ShapeDtypeStruct((n, d), jnp.float32))
    tc_post = pl.pallas_call(
        _tc_post_body,
        out_shape=jax.ShapeDtypeStruct((n, d), jnp.float32))
    return deg_call, agg_call, tc_pre, tc_mid, tc_post, ncd, nca, npad


def _pad_per_worker(a, slots, epw, padvals):
    return jnp.concatenate([a.reshape(NW, epw), padvals], axis=1)


def kernel(x, edge_index, edge_weight, W1, b1, W2, b2):
    n, d_in = x.shape
    e = edge_weight.shape[0]
    d = W1.shape[0]
    (deg_call, agg_call, tc_pre, tc_mid, tc_post,
     ncd, nca, npad) = _make_calls(n, d, e)

    epw = e // NW
    row = edge_index[0]
    col = edge_index[1]
    ew = edge_weight
    padd = ncd * CH - epw
    pada = (nca + 2) * CHA - epw
    pidxd = (jnp.arange(NW * padd, dtype=jnp.int32) % n).reshape(NW, padd)
    pidxa = (jnp.arange(NW * pada, dtype=jnp.int32) % n).reshape(NW, pada)
    zpad = jnp.zeros((NW, padd), jnp.float32)
    zpaa = jnp.zeros((NW, pada), jnp.float32)

    col_pd = _pad_per_worker(col, ncd, epw, pidxd).reshape(NW, ncd, CH)
    ew_pd = _pad_per_worker(ew, ncd, epw, zpad).reshape(NW, ncd, CH)
    rowf = _pad_per_worker(row, nca, epw, pidxa).reshape(-1)
    colf = _pad_per_worker(col, nca, epw, pidxa).reshape(-1)
    ewf = _pad_per_worker(ew, nca, epw, zpaa).reshape(-1)

    zn = jnp.zeros((n,), jnp.float32)
    znd = jnp.zeros((npad, d), jnp.float32)

    deg_part = deg_call(col_pd, ew_pd, zn)
    dinv, h1s = tc_pre(x, W1.T, deg_part.reshape(NC, n).T)
    agg1 = agg_call(h1s, rowf, colf, ewf, znd)
    h2s = tc_mid(agg1, h1s, dinv, b1.reshape(1, d), W2.T)
    agg2 = agg_call(h2s, rowf, colf, ewf, znd)
    out = tc_post(agg2, h2s, dinv, b2.reshape(1, d))
    return out
